# Initial kernel scaffold; baseline (speedup 1.0000x reference)
import jax
import jax.numpy as jnp
from jax.experimental import pallas as pl


def kernel(x, edge_index, edge_type, category_idx, gt_weight, channel_params, W1, b1, W2, b2):
    raise NotImplementedError("write your pallas kernel here")


# speedup vs baseline: 9.8330x; 9.8330x over previous
import functools

import jax
import jax.numpy as jnp
from jax import lax
from jax.experimental import pallas as pl
from jax.experimental.pallas import tpu as pltpu
from jax.experimental.pallas import tpu_sc as plsc

N = 10000
E = 320000
NUM_CHANNELS = 2
IN_DIM = 128
HIDDEN = 64
NUM_CLASS = 16
N_CAT = 2000

NC = 2
NS = 16
CHUNK = 128
TILE_E = 20096
E_PAD = TILE_E * NS
NROW = 10240
N_TILE = NROW // NS
ROW_SUB = 64
AW = 80
HROW = 128
CAT_PAD = 2048


def _zero16():
    return jnp.zeros((16,), jnp.float32)


def _layer_body(hsrc, src2, dstp, etp, filt, hout,
                acc, src_v, dst_v, et_v, filt_v, rows, scaled,
                accbuf, hbuf, outbuf, sem):
    c = lax.axis_index("c")
    s = lax.axis_index("s")

    pltpu.sync_copy(filt.at[c], filt_v)

    z16 = _zero16()
    for r in range(ROW_SUB):
        for k in range(AW // 16):
            accbuf[r, pl.ds(k * 16, 16)] = z16
        for k in range(4, 8):
            outbuf[r, pl.ds(k * 16, 16)] = z16

    def zero_m(m, carry):
        r0 = s * N_TILE + m * ROW_SUB
        pltpu.sync_copy(accbuf.at[pl.ds(0, ROW_SUB)], acc.at[pl.ds(r0, ROW_SUB)])
        return carry
    lax.fori_loop(0, N_TILE // ROW_SUB, zero_m, 0)
    plsc.subcore_barrier()

    e_base = s * TILE_E

    def edge_chunk(i, carry):
        base = e_base + i * CHUNK
        pltpu.sync_copy(src2.at[c, pl.ds(base, CHUNK)], src_v)
        pltpu.sync_copy(dstp.at[pl.ds(base, CHUNK)], dst_v)
        pltpu.sync_copy(etp.at[pl.ds(base, CHUNK)], et_v)
        pltpu.async_copy(hsrc.at[src_v], rows, sem).wait()
        e0 = jnp.where(lax.iota(jnp.int32, 16) == 0, 1.0, 0.0)
        for g in range(CHUNK // 16):
            et16 = et_v[pl.ds(g * 16, 16)]
            w16 = plsc.load_gather(filt_v, [et16])
            for j in range(16):
                e = g * 16 + j
                w = w16[j]
                for k in range(4):
                    scaled[e, pl.ds(k * 16, 16)] = rows[e, pl.ds(k * 16, 16)] * w
                scaled[e, pl.ds(64, 16)] = e0 * w
        pltpu.sync_copy(scaled, acc.at[dst_v], add=True)
        return carry
    lax.fori_loop(0, TILE_E // CHUNK, edge_chunk, 0)
    plsc.subcore_barrier()

    f4 = filt_v[...][4]
    col64 = jnp.full((16,), 64, jnp.int32)

    def norm_m(m, carry):
        r0 = s * N_TILE + m * ROW_SUB
        pltpu.sync_copy(acc.at[pl.ds(r0, ROW_SUB)], accbuf.at[pl.ds(0, ROW_SUB)])
        pltpu.sync_copy(hsrc.at[pl.ds(c * NROW + r0, ROW_SUB)],
                        hbuf.at[pl.ds(0, ROW_SUB)])
        for g in range(ROW_SUB // 16):
            rowids = lax.iota(jnp.int32, 16) + g * 16
            d16 = plsc.load_gather(accbuf, [rowids, col64]) + f4
            rec16 = 1.0 / d16
            for j in range(16):
                r = g * 16 + j
                rec = rec16[j]
                f4rec = f4 * rec
                for k in range(4):
                    sl = pl.ds(k * 16, 16)
                    outbuf[r, sl] = accbuf[r, sl] * rec + hbuf[r, sl] * f4rec
        pltpu.sync_copy(outbuf.at[pl.ds(0, ROW_SUB)],
                        hout.at[pl.ds(c * NROW + r0, ROW_SUB)])
        return carry
    lax.fori_loop(0, N_TILE // ROW_SUB, norm_m, 0)


def _sc_layer(hsrc, src2, dstp, etp, filt):
    mesh = plsc.VectorSubcoreMesh(core_axis_name="c", subcore_axis_name="s",
                                  num_cores=NC, num_subcores=NS)
    return pl.kernel(
        _layer_body,
        out_type=jax.ShapeDtypeStruct((NC * NROW, HROW), jnp.float32),
        mesh=mesh,
        scratch_types=[
            pltpu.VMEM_SHARED((NROW, AW), jnp.float32),
            pltpu.VMEM((CHUNK,), jnp.int32),
            pltpu.VMEM((CHUNK,), jnp.int32),
            pltpu.VMEM((CHUNK,), jnp.int32),
            pltpu.VMEM((16,), jnp.float32),
            pltpu.VMEM((CHUNK, HROW), jnp.float32),
            pltpu.VMEM((CHUNK, AW), jnp.float32),
            pltpu.VMEM((ROW_SUB, AW), jnp.float32),
            pltpu.VMEM((ROW_SUB, HROW), jnp.float32),
            pltpu.VMEM((ROW_SUB, HROW), jnp.float32),
            pltpu.SemaphoreType.DMA,
        ],
        compiler_params=pltpu.CompilerParams(needs_layout_passes=False),
        name="fastgtn_layer",
    )(hsrc, src2, dstp, etp, filt)


def _gather_body(hfin, catp, out, cat_v, rowsbuf, sem):
    c = lax.axis_index("c")
    s = lax.axis_index("s")
    per = CAT_PAD // NS
    base = s * per
    pltpu.sync_copy(catp.at[pl.ds(base, per)], cat_v)
    shift = c * NROW
    for g in range(per // 16):
        sl = pl.ds(g * 16, 16)
        cat_v[sl] = cat_v[sl] + shift
    pltpu.async_copy(hfin.at[cat_v], rowsbuf, sem).wait()
    pltpu.sync_copy(rowsbuf, out.at[c, pl.ds(base, per)])


def _sc_gather_cat(hfin, catp):
    mesh = plsc.VectorSubcoreMesh(core_axis_name="c", subcore_axis_name="s",
                                  num_cores=NC, num_subcores=NS)
    per = CAT_PAD // NS
    return pl.kernel(
        _gather_body,
        out_type=jax.ShapeDtypeStruct((NC, CAT_PAD, HROW), jnp.float32),
        mesh=mesh,
        scratch_types=[
            pltpu.VMEM((per,), jnp.int32),
            pltpu.VMEM((per, HROW), jnp.float32),
            pltpu.SemaphoreType.DMA,
        ],
        name="fastgtn_cat_gather",
    )(hfin, catp)


def _proj_body(x_ref, cp_ref, o_ref):
    h = jnp.dot(x_ref[...], cp_ref[0], preferred_element_type=jnp.float32)
    o_ref[...] = jnp.concatenate(
        [h, jnp.zeros_like(h)], axis=1)


def _tc_proj(x, cp):
    nb = 10
    bs = NROW // nb
    return pl.pallas_call(
        _proj_body,
        grid=(NUM_CHANNELS, nb),
        in_specs=[
            pl.BlockSpec((bs, IN_DIM), lambda c, i: (i, 0)),
            pl.BlockSpec((1, IN_DIM, HIDDEN), lambda c, i: (c, 0, 0)),
        ],
        out_specs=pl.BlockSpec((bs, HROW), lambda c, i: (c * nb + i, 0)),
        out_shape=jax.ShapeDtypeStruct((NC * NROW, HROW), jnp.float32),
    )(x, cp)


def _mlp_body(a_ref, w1_ref, b1_ref, w2_ref, b2_ref, o_ref):
    h = jnp.dot(a_ref[0, :, :HIDDEN], w1_ref[0],
                preferred_element_type=jnp.float32)
    h = h + jnp.dot(a_ref[1, :, :HIDDEN], w1_ref[1],
                    preferred_element_type=jnp.float32)
    h = jnp.maximum(h + b1_ref[...], 0.0)
    o_ref[...] = jnp.dot(h, w2_ref[...],
                         preferred_element_type=jnp.float32) + b2_ref[...]


def _tc_mlp(a, w1s, b1, w2, b2):
    nb = 8
    bs = CAT_PAD // nb
    return pl.pallas_call(
        _mlp_body,
        grid=(nb,),
        in_specs=[
            pl.BlockSpec((NC, bs, HROW), lambda i: (0, i, 0)),
            pl.BlockSpec((NC, HIDDEN, HIDDEN), lambda i: (0, 0, 0)),
            pl.BlockSpec((1, HIDDEN), lambda i: (0, 0)),
            pl.BlockSpec((HIDDEN, NUM_CLASS), lambda i: (0, 0)),
            pl.BlockSpec((1, NUM_CLASS), lambda i: (0, 0)),
        ],
        out_specs=pl.BlockSpec((bs, NUM_CLASS), lambda i: (i, 0)),
        out_shape=jax.ShapeDtypeStruct((CAT_PAD, NUM_CLASS), jnp.float32),
    )(a, w1s, b1, w2, b2)


def kernel(x, edge_index, edge_type, category_idx, gt_weight, channel_params,
           W1, b1, W2, b2):
    filt_all = jax.nn.softmax(gt_weight, axis=-1)
    nl = filt_all.shape[0]
    filt_tabs = jnp.zeros((nl, NUM_CHANNELS, 16), jnp.float32)
    filt_tabs = filt_tabs.at[:, :, :5].set(filt_all)

    npad = E_PAD - E
    spread = (jnp.arange(npad, dtype=jnp.int32) * 7) % N
    src_p = jnp.concatenate([edge_index[0], spread])
    dst_p = jnp.concatenate([edge_index[1], spread])
    et_p = jnp.concatenate([edge_type, jnp.full((npad,), 5, jnp.int32)])
    src2 = jnp.stack([src_p, src_p + NROW])

    cat_p = jnp.concatenate([
        category_idx,
        jnp.zeros((CAT_PAD - N_CAT,), jnp.int32)])

    x_pad = jnp.pad(x, ((0, NROW - N), (0, 0)))
    h = _tc_proj(x_pad, channel_params)
    for i in range(nl):
        h = _sc_layer(h, src2, dst_p, et_p, filt_tabs[i])
    a = _sc_gather_cat(h, cat_p)
    y = _tc_mlp(a, W1.reshape(NC, HIDDEN, HIDDEN),
                b1.reshape(1, HIDDEN), W2, b2.reshape(1, NUM_CLASS))
    return y[:N_CAT]


# speedup vs baseline: 12.5192x; 1.2732x over previous
import functools

import jax
import jax.numpy as jnp
from jax import lax
from jax.experimental import pallas as pl
from jax.experimental.pallas import tpu as pltpu
from jax.experimental.pallas import tpu_sc as plsc

N = 10000
E = 320000
NUM_CHANNELS = 2
IN_DIM = 128
HIDDEN = 64
NUM_CLASS = 16
N_CAT = 2000

NC = 2
NS = 16
CHUNK = 128
TILE_E = 20224
E_PAD = TILE_E * NS
NROW = 10240
N_TILE = NROW // NS
ROW_SUB = 64
AW = 80
HROW = 128
CAT_PAD = 2048


def _zero16():
    return jnp.zeros((16,), jnp.float32)


def _layer_body(hsrc, e3, filt, hout,
                acc, e3b, src_v, dst_v, filt_v, rows, scaled,
                accbuf, hbuf, outbuf, sem):
    c = lax.axis_index("c")
    s = lax.axis_index("s")

    pltpu.sync_copy(filt.at[c], filt_v)

    z16 = _zero16()
    for r in range(ROW_SUB):
        for k in range(AW // 16):
            accbuf[r, pl.ds(k * 16, 16)] = z16
        for k in range(4, 8):
            outbuf[r, pl.ds(k * 16, 16)] = z16
    for r in range(CHUNK):
        scaled[r, pl.ds(64, 16)] = z16

    def zero_m(m, carry):
        r0 = s * N_TILE + m * ROW_SUB
        pltpu.sync_copy(accbuf.at[pl.ds(0, ROW_SUB)], acc.at[pl.ds(r0, ROW_SUB)])
        return carry
    lax.fori_loop(0, N_TILE // ROW_SUB, zero_m, 0)
    plsc.subcore_barrier()

    e_base = s * TILE_E

    shift = c * NROW

    def edge_chunk(i, carry):
        cid = s * (TILE_E // CHUNK) + i
        pltpu.sync_copy(e3.at[cid], e3b)
        for g in range(CHUNK // 16):
            sl = pl.ds(g * 16, 16)
            src_v[sl] = e3b[0, sl] + shift
            dst_v[sl] = e3b[1, sl]
        pltpu.async_copy(hsrc.at[src_v], rows, sem).wait()
        for g in range(CHUNK // 16):
            et16 = e3b[2, pl.ds(g * 16, 16)]
            w16 = plsc.load_gather(filt_v, [et16])
            rowids = lax.iota(jnp.int32, 16) + g * 16
            plsc.store_scatter(scaled, [rowids, jnp.full((16,), 64, jnp.int32)],
                               w16)
            for j in range(16):
                e = g * 16 + j
                w = w16[j]
                for k in range(4):
                    scaled[e, pl.ds(k * 16, 16)] = rows[e, pl.ds(k * 16, 16)] * w
        pltpu.sync_copy(scaled, acc.at[dst_v], add=True)
        return carry
    lax.fori_loop(0, TILE_E // CHUNK, edge_chunk, 0)
    plsc.subcore_barrier()

    f4 = filt_v[...][4]
    col64 = jnp.full((16,), 64, jnp.int32)

    def norm_m(m, carry):
        r0 = s * N_TILE + m * ROW_SUB
        pltpu.sync_copy(acc.at[pl.ds(r0, ROW_SUB)], accbuf.at[pl.ds(0, ROW_SUB)])
        pltpu.sync_copy(hsrc.at[pl.ds(c * NROW + r0, ROW_SUB)],
                        hbuf.at[pl.ds(0, ROW_SUB)])
        for g in range(ROW_SUB // 16):
            rowids = lax.iota(jnp.int32, 16) + g * 16
            d16 = plsc.load_gather(accbuf, [rowids, col64]) + f4
            rec16 = 1.0 / d16
            for j in range(16):
                r = g * 16 + j
                rec = rec16[j]
                f4rec = f4 * rec
                for k in range(4):
                    sl = pl.ds(k * 16, 16)
                    outbuf[r, sl] = accbuf[r, sl] * rec + hbuf[r, sl] * f4rec
        pltpu.sync_copy(outbuf.at[pl.ds(0, ROW_SUB)],
                        hout.at[pl.ds(c * NROW + r0, ROW_SUB)])
        return carry
    lax.fori_loop(0, N_TILE // ROW_SUB, norm_m, 0)


def _sc_layer(hsrc, e3, filt):
    mesh = plsc.VectorSubcoreMesh(core_axis_name="c", subcore_axis_name="s",
                                  num_cores=NC, num_subcores=NS)
    return pl.kernel(
        _layer_body,
        out_type=jax.ShapeDtypeStruct((NC * NROW, HROW), jnp.float32),
        mesh=mesh,
        scratch_types=[
            pltpu.VMEM_SHARED((NROW, AW), jnp.float32),
            pltpu.VMEM((3, CHUNK), jnp.int32),
            pltpu.VMEM((CHUNK,), jnp.int32),
            pltpu.VMEM((CHUNK,), jnp.int32),
            pltpu.VMEM((16,), jnp.float32),
            pltpu.VMEM((CHUNK, HROW), jnp.float32),
            pltpu.VMEM((CHUNK, AW), jnp.float32),
            pltpu.VMEM((ROW_SUB, AW), jnp.float32),
            pltpu.VMEM((ROW_SUB, HROW), jnp.float32),
            pltpu.VMEM((ROW_SUB, HROW), jnp.float32),
            pltpu.SemaphoreType.DMA,
        ],
        compiler_params=pltpu.CompilerParams(needs_layout_passes=False),
        name="fastgtn_layer",
    )(hsrc, e3, filt)


def _gather_body(hfin, catp, out, cat_v, rowsbuf, sem):
    c = lax.axis_index("c")
    s = lax.axis_index("s")
    per = CAT_PAD // NS
    base = s * per
    pltpu.sync_copy(catp.at[pl.ds(base, per)], cat_v)
    shift = c * NROW
    for g in range(per // 16):
        sl = pl.ds(g * 16, 16)
        cat_v[sl] = cat_v[sl] + shift
    pltpu.async_copy(hfin.at[cat_v], rowsbuf, sem).wait()
    pltpu.sync_copy(rowsbuf, out.at[c, pl.ds(base, per)])


def _sc_gather_cat(hfin, catp):
    mesh = plsc.VectorSubcoreMesh(core_axis_name="c", subcore_axis_name="s",
                                  num_cores=NC, num_subcores=NS)
    per = CAT_PAD // NS
    return pl.kernel(
        _gather_body,
        out_type=jax.ShapeDtypeStruct((NC, CAT_PAD, HROW), jnp.float32),
        mesh=mesh,
        scratch_types=[
            pltpu.VMEM((per,), jnp.int32),
            pltpu.VMEM((per, HROW), jnp.float32),
            pltpu.SemaphoreType.DMA,
        ],
        name="fastgtn_cat_gather",
    )(hfin, catp)


def _proj_body(x_ref, cp_ref, o_ref):
    h = jnp.dot(x_ref[...], cp_ref[0], preferred_element_type=jnp.float32)
    o_ref[...] = jnp.concatenate(
        [h, jnp.zeros_like(h)], axis=1)


def _tc_proj(x, cp):
    nb = 10
    bs = NROW // nb
    return pl.pallas_call(
        _proj_body,
        grid=(NUM_CHANNELS, nb),
        in_specs=[
            pl.BlockSpec((bs, IN_DIM), lambda c, i: (i, 0)),
            pl.BlockSpec((1, IN_DIM, HIDDEN), lambda c, i: (c, 0, 0)),
        ],
        out_specs=pl.BlockSpec((bs, HROW), lambda c, i: (c * nb + i, 0)),
        out_shape=jax.ShapeDtypeStruct((NC * NROW, HROW), jnp.float32),
    )(x, cp)


def _mlp_body(a_ref, w1_ref, b1_ref, w2_ref, b2_ref, o_ref):
    h = jnp.dot(a_ref[0, :, :HIDDEN], w1_ref[0],
                preferred_element_type=jnp.float32)
    h = h + jnp.dot(a_ref[1, :, :HIDDEN], w1_ref[1],
                    preferred_element_type=jnp.float32)
    h = jnp.maximum(h + b1_ref[...], 0.0)
    o_ref[...] = jnp.dot(h, w2_ref[...],
                         preferred_element_type=jnp.float32) + b2_ref[...]


def _tc_mlp(a, w1s, b1, w2, b2):
    nb = 8
    bs = CAT_PAD // nb
    return pl.pallas_call(
        _mlp_body,
        grid=(nb,),
        in_specs=[
            pl.BlockSpec((NC, bs, HROW), lambda i: (0, i, 0)),
            pl.BlockSpec((NC, HIDDEN, HIDDEN), lambda i: (0, 0, 0)),
            pl.BlockSpec((1, HIDDEN), lambda i: (0, 0)),
            pl.BlockSpec((HIDDEN, NUM_CLASS), lambda i: (0, 0)),
            pl.BlockSpec((1, NUM_CLASS), lambda i: (0, 0)),
        ],
        out_specs=pl.BlockSpec((bs, NUM_CLASS), lambda i: (i, 0)),
        out_shape=jax.ShapeDtypeStruct((CAT_PAD, NUM_CLASS), jnp.float32),
    )(a, w1s, b1, w2, b2)


def kernel(x, edge_index, edge_type, category_idx, gt_weight, channel_params,
           W1, b1, W2, b2):
    filt_all = jax.nn.softmax(gt_weight, axis=-1)
    nl = filt_all.shape[0]
    filt_tabs = jnp.zeros((nl, NUM_CHANNELS, 16), jnp.float32)
    filt_tabs = filt_tabs.at[:, :, :5].set(filt_all)

    npad = E_PAD - E
    spread = (jnp.arange(npad, dtype=jnp.int32) * 7) % N
    src_p = jnp.concatenate([edge_index[0], spread])
    dst_p = jnp.concatenate([edge_index[1], spread])
    et_p = jnp.concatenate([edge_type, jnp.full((npad,), 5, jnp.int32)])
    edges3 = jnp.stack([src_p.reshape(-1, CHUNK), dst_p.reshape(-1, CHUNK),
                        et_p.reshape(-1, CHUNK)], axis=1)

    cat_p = jnp.concatenate([
        category_idx,
        jnp.zeros((CAT_PAD - N_CAT,), jnp.int32)])

    x_pad = jnp.pad(x, ((0, NROW - N), (0, 0)))
    h = _tc_proj(x_pad, channel_params)
    for i in range(nl):
        h = _sc_layer(h, edges3, filt_tabs[i])
    a = _sc_gather_cat(h, cat_p)
    y = _tc_mlp(a, W1.reshape(NC, HIDDEN, HIDDEN),
                b1.reshape(1, HIDDEN), W2, b2.reshape(1, NUM_CLASS))
    return y[:N_CAT]


# speedup vs baseline: 14.3806x; 1.1487x over previous
import functools

import jax
import jax.numpy as jnp
from jax import lax
from jax.experimental import pallas as pl
from jax.experimental.pallas import tpu as pltpu
from jax.experimental.pallas import tpu_sc as plsc

N = 10000
E = 320000
NUM_CHANNELS = 2
IN_DIM = 128
HIDDEN = 64
NUM_CLASS = 16
N_CAT = 2000

NC = 2
NS = 16
CHUNK = 256
TILE_E = 20480
E_PAD = TILE_E * NS
NROW = 10240
N_TILE = NROW // NS
ROW_SUB = 16
AW = 80
HROW = 128
CAT_PAD = 2048


def _zero16():
    return jnp.zeros((16,), jnp.float32)


def _layer_body(hsrc, e3, filt, hout,
                acc, e3b, src_v, dst_v, filt_v, rows, scaled,
                accbuf, hbuf, outbuf, sem):
    c = lax.axis_index("c")
    s = lax.axis_index("s")

    pltpu.sync_copy(filt.at[c], filt_v)

    z16 = _zero16()
    for r in range(ROW_SUB):
        for k in range(AW // 16):
            accbuf[r, pl.ds(k * 16, 16)] = z16
        for k in range(4, 8):
            outbuf[r, pl.ds(k * 16, 16)] = z16
    for r in range(CHUNK):
        scaled[r, pl.ds(64, 16)] = z16

    def zero_m(m, carry):
        r0 = s * N_TILE + m * ROW_SUB
        pltpu.sync_copy(accbuf.at[pl.ds(0, ROW_SUB)], acc.at[pl.ds(r0, ROW_SUB)])
        return carry
    lax.fori_loop(0, N_TILE // ROW_SUB, zero_m, 0)
    plsc.subcore_barrier()

    e_base = s * TILE_E

    shift = c * NROW

    def edge_chunk(i, carry):
        cid = s * (TILE_E // CHUNK) + i
        pltpu.sync_copy(e3.at[cid], e3b)
        for g in range(CHUNK // 16):
            sl = pl.ds(g * 16, 16)
            src_v[sl] = e3b[0, sl] + shift
            dst_v[sl] = e3b[1, sl]
        pltpu.async_copy(hsrc.at[src_v], rows, sem).wait()
        for g in range(CHUNK // 16):
            et16 = e3b[2, pl.ds(g * 16, 16)]
            w16 = plsc.load_gather(filt_v, [et16])
            rowids = lax.iota(jnp.int32, 16) + g * 16
            plsc.store_scatter(scaled, [rowids, jnp.full((16,), 64, jnp.int32)],
                               w16)
            for j in range(16):
                e = g * 16 + j
                w = w16[j]
                for k in range(4):
                    scaled[e, pl.ds(k * 16, 16)] = rows[e, pl.ds(k * 16, 16)] * w
        pltpu.sync_copy(scaled, acc.at[dst_v], add=True)
        return carry
    lax.fori_loop(0, TILE_E // CHUNK, edge_chunk, 0)
    plsc.subcore_barrier()

    f4 = filt_v[...][4]
    col64 = jnp.full((16,), 64, jnp.int32)

    def norm_m(m, carry):
        r0 = s * N_TILE + m * ROW_SUB
        pltpu.sync_copy(acc.at[pl.ds(r0, ROW_SUB)], accbuf.at[pl.ds(0, ROW_SUB)])
        pltpu.sync_copy(hsrc.at[pl.ds(c * NROW + r0, ROW_SUB)],
                        hbuf.at[pl.ds(0, ROW_SUB)])
        for g in range(ROW_SUB // 16):
            rowids = lax.iota(jnp.int32, 16) + g * 16
            d16 = plsc.load_gather(accbuf, [rowids, col64]) + f4
            rec16 = 1.0 / d16
            for j in range(16):
                r = g * 16 + j
                rec = rec16[j]
                f4rec = f4 * rec
                for k in range(4):
                    sl = pl.ds(k * 16, 16)
                    outbuf[r, sl] = accbuf[r, sl] * rec + hbuf[r, sl] * f4rec
        pltpu.sync_copy(outbuf.at[pl.ds(0, ROW_SUB)],
                        hout.at[pl.ds(c * NROW + r0, ROW_SUB)])
        return carry
    lax.fori_loop(0, N_TILE // ROW_SUB, norm_m, 0)


def _sc_layer(hsrc, e3, filt):
    mesh = plsc.VectorSubcoreMesh(core_axis_name="c", subcore_axis_name="s",
                                  num_cores=NC, num_subcores=NS)
    return pl.kernel(
        _layer_body,
        out_type=jax.ShapeDtypeStruct((NC * NROW, HROW), jnp.float32),
        mesh=mesh,
        scratch_types=[
            pltpu.VMEM_SHARED((NROW, AW), jnp.float32),
            pltpu.VMEM((3, CHUNK), jnp.int32),
            pltpu.VMEM((CHUNK,), jnp.int32),
            pltpu.VMEM((CHUNK,), jnp.int32),
            pltpu.VMEM((16,), jnp.float32),
            pltpu.VMEM((CHUNK, HROW), jnp.float32),
            pltpu.VMEM((CHUNK, AW), jnp.float32),
            pltpu.VMEM((ROW_SUB, AW), jnp.float32),
            pltpu.VMEM((ROW_SUB, HROW), jnp.float32),
            pltpu.VMEM((ROW_SUB, HROW), jnp.float32),
            pltpu.SemaphoreType.DMA,
        ],
        compiler_params=pltpu.CompilerParams(needs_layout_passes=False),
        name="fastgtn_layer",
    )(hsrc, e3, filt)


def _gather_body(hfin, catp, out, cat_v, rowsbuf, sem):
    c = lax.axis_index("c")
    s = lax.axis_index("s")
    per = CAT_PAD // NS
    base = s * per
    pltpu.sync_copy(catp.at[pl.ds(base, per)], cat_v)
    shift = c * NROW
    for g in range(per // 16):
        sl = pl.ds(g * 16, 16)
        cat_v[sl] = cat_v[sl] + shift
    pltpu.async_copy(hfin.at[cat_v], rowsbuf, sem).wait()
    pltpu.sync_copy(rowsbuf, out.at[c, pl.ds(base, per)])


def _sc_gather_cat(hfin, catp):
    mesh = plsc.VectorSubcoreMesh(core_axis_name="c", subcore_axis_name="s",
                                  num_cores=NC, num_subcores=NS)
    per = CAT_PAD // NS
    return pl.kernel(
        _gather_body,
        out_type=jax.ShapeDtypeStruct((NC, CAT_PAD, HROW), jnp.float32),
        mesh=mesh,
        scratch_types=[
            pltpu.VMEM((per,), jnp.int32),
            pltpu.VMEM((per, HROW), jnp.float32),
            pltpu.SemaphoreType.DMA,
        ],
        name="fastgtn_cat_gather",
    )(hfin, catp)


def _proj_body(x_ref, cp_ref, o_ref):
    h = jnp.dot(x_ref[...], cp_ref[0], preferred_element_type=jnp.float32)
    o_ref[...] = jnp.concatenate(
        [h, jnp.zeros_like(h)], axis=1)


def _tc_proj(x, cp):
    nb = 10
    bs = NROW // nb
    return pl.pallas_call(
        _proj_body,
        grid=(NUM_CHANNELS, nb),
        in_specs=[
            pl.BlockSpec((bs, IN_DIM), lambda c, i: (i, 0)),
            pl.BlockSpec((1, IN_DIM, HIDDEN), lambda c, i: (c, 0, 0)),
        ],
        out_specs=pl.BlockSpec((bs, HROW), lambda c, i: (c * nb + i, 0)),
        out_shape=jax.ShapeDtypeStruct((NC * NROW, HROW), jnp.float32),
    )(x, cp)


def _mlp_body(a_ref, w1_ref, b1_ref, w2_ref, b2_ref, o_ref):
    h = jnp.dot(a_ref[0, :, :HIDDEN], w1_ref[0],
                preferred_element_type=jnp.float32)
    h = h + jnp.dot(a_ref[1, :, :HIDDEN], w1_ref[1],
                    preferred_element_type=jnp.float32)
    h = jnp.maximum(h + b1_ref[...], 0.0)
    o_ref[...] = jnp.dot(h, w2_ref[...],
                         preferred_element_type=jnp.float32) + b2_ref[...]


def _tc_mlp(a, w1s, b1, w2, b2):
    nb = 8
    bs = CAT_PAD // nb
    return pl.pallas_call(
        _mlp_body,
        grid=(nb,),
        in_specs=[
            pl.BlockSpec((NC, bs, HROW), lambda i: (0, i, 0)),
            pl.BlockSpec((NC, HIDDEN, HIDDEN), lambda i: (0, 0, 0)),
            pl.BlockSpec((1, HIDDEN), lambda i: (0, 0)),
            pl.BlockSpec((HIDDEN, NUM_CLASS), lambda i: (0, 0)),
            pl.BlockSpec((1, NUM_CLASS), lambda i: (0, 0)),
        ],
        out_specs=pl.BlockSpec((bs, NUM_CLASS), lambda i: (i, 0)),
        out_shape=jax.ShapeDtypeStruct((CAT_PAD, NUM_CLASS), jnp.float32),
    )(a, w1s, b1, w2, b2)


def kernel(x, edge_index, edge_type, category_idx, gt_weight, channel_params,
           W1, b1, W2, b2):
    filt_all = jax.nn.softmax(gt_weight, axis=-1)
    nl = filt_all.shape[0]
    filt_tabs = jnp.zeros((nl, NUM_CHANNELS, 16), jnp.float32)
    filt_tabs = filt_tabs.at[:, :, :5].set(filt_all)

    npad = E_PAD - E
    spread = (jnp.arange(npad, dtype=jnp.int32) * 7) % N
    src_p = jnp.concatenate([edge_index[0], spread])
    dst_p = jnp.concatenate([edge_index[1], spread])
    et_p = jnp.concatenate([edge_type, jnp.full((npad,), 5, jnp.int32)])
    edges3 = jnp.stack([src_p.reshape(-1, CHUNK), dst_p.reshape(-1, CHUNK),
                        et_p.reshape(-1, CHUNK)], axis=1)

    cat_p = jnp.concatenate([
        category_idx,
        jnp.zeros((CAT_PAD - N_CAT,), jnp.int32)])

    x_pad = jnp.pad(x, ((0, NROW - N), (0, 0)))
    h = _tc_proj(x_pad, channel_params)
    for i in range(nl):
        h = _sc_layer(h, edges3, filt_tabs[i])
    a = _sc_gather_cat(h, cat_p)
    y = _tc_mlp(a, W1.reshape(NC, HIDDEN, HIDDEN),
                b1.reshape(1, HIDDEN), W2, b2.reshape(1, NUM_CLASS))
    return y[:N_CAT]
